# baseline (speedup 1.0000x reference)
import functools

import jax
import jax.numpy as jnp
from jax import lax
from jax.experimental import pallas as pl
from jax.experimental.pallas import tpu as pltpu
from jax.experimental.pallas import tpu_sc as plsc

_N = 524288
_NC = 1
_NS = 16
_NW = _NC * _NS
_L = 16
_PER_W = _N // _NW
_NCHUNK = 1
_CHUNK = _PER_W // _NCHUNK
_UNROLL = 8


def _build_sc_clamp():
    mesh = plsc.VectorSubcoreMesh(core_axis_name="c", subcore_axis_name="s", num_cores=1)

    @functools.partial(
        pl.kernel,
        mesh=mesh,
        out_type=jax.ShapeDtypeStruct((_N,), jnp.float32),
        scratch_types=[
            pltpu.VMEM((_PER_W,), jnp.float32),
            pltpu.VMEM((64,), jnp.float32),
        ] + [pltpu.SemaphoreType.DMA] * (_NCHUNK + 2),
    )
    def sc_clamp(x_hbm, vals_hbm, out_hbm, buf, vals_v, *sems):
        in_sems, vsem, osem = sems[:_NCHUNK], sems[_NCHUNK], sems[_NCHUNK + 1]
        wid = lax.axis_index("s") * _NC + lax.axis_index("c")
        base = wid * _PER_W
        vcopy = pltpu.async_copy(vals_hbm, vals_v, vsem)
        in_copies = []
        for c in range(_NCHUNK):
            off = c * _CHUNK
            in_copies.append(pltpu.async_copy(
                x_hbm.at[pl.ds(base + off, _CHUNK)],
                buf.at[pl.ds(off, _CHUNK)], in_sems[c]))
        vcopy.wait()
        lo = jnp.full((_L,), vals_v[pl.ds(0, _L)][0], jnp.float32)
        hi = jnp.full((_L,), vals_v[pl.ds(48, _L)][_L - 1], jnp.float32)

        out_copies = []
        for c in range(_NCHUNK):
            off = c * _CHUNK
            in_copies[c].wait()

            def body(i, carry, off=off):
                o = off + i * (_L * _UNROLL)
                for j in range(_UNROLL):
                    s = pl.ds(o + j * _L, _L)
                    buf[s] = jnp.maximum(jnp.minimum(buf[s], hi), lo)
                return carry

            lax.fori_loop(0, _CHUNK // (_L * _UNROLL), body, 0)
            out_copies.append(pltpu.async_copy(
                buf.at[pl.ds(off, _CHUNK)],
                out_hbm.at[pl.ds(base + off, _CHUNK)], osem))
        for cp in out_copies:
            cp.wait()

    return sc_clamp


_sc_clamp = _build_sc_clamp()


def kernel(tensor_input, values):
    out = _sc_clamp(tensor_input.reshape(_N), values)
    return out.reshape(tensor_input.shape)


# speedup vs baseline: 1.0265x; 1.0265x over previous
import functools

import jax
import jax.numpy as jnp
from jax import lax
from jax.experimental import pallas as pl
from jax.experimental.pallas import tpu as pltpu
from jax.experimental.pallas import tpu_sc as plsc

_N = 524288
_NC = 1
_NS = 16
_NW = _NC * _NS
_L = 16
_PER_W = _N // _NW
_NCHUNK = 4
_CHUNK = _PER_W // _NCHUNK
_UNROLL = 16


def _build_sc_clamp():
    mesh = plsc.VectorSubcoreMesh(core_axis_name="c", subcore_axis_name="s", num_cores=1)

    @functools.partial(
        pl.kernel,
        mesh=mesh,
        out_type=jax.ShapeDtypeStruct((_N,), jnp.float32),
        scratch_types=[
            pltpu.VMEM((_PER_W,), jnp.float32),
            pltpu.VMEM((64,), jnp.float32),
        ] + [pltpu.SemaphoreType.DMA] * (_NCHUNK + 2),
    )
    def sc_clamp(x_hbm, vals_hbm, out_hbm, buf, vals_v, *sems):
        in_sems, vsem, osem = sems[:_NCHUNK], sems[_NCHUNK], sems[_NCHUNK + 1]
        wid = lax.axis_index("s") * _NC + lax.axis_index("c")
        base = wid * _PER_W
        vcopy = pltpu.async_copy(vals_hbm, vals_v, vsem)
        in_copies = []
        for c in range(_NCHUNK):
            off = c * _CHUNK
            in_copies.append(pltpu.async_copy(
                x_hbm.at[pl.ds(base + off, _CHUNK)],
                buf.at[pl.ds(off, _CHUNK)], in_sems[c]))
        vcopy.wait()
        lo = jnp.full((_L,), vals_v[pl.ds(0, _L)][0], jnp.float32)
        hi = jnp.full((_L,), vals_v[pl.ds(48, _L)][_L - 1], jnp.float32)

        out_copies = []
        for c in range(_NCHUNK):
            off = c * _CHUNK
            in_copies[c].wait()

            def body(i, carry, off=off):
                o = off + i * (_L * _UNROLL)
                for j in range(_UNROLL):
                    s = pl.ds(o + j * _L, _L)
                    buf[s] = jnp.maximum(jnp.minimum(buf[s], hi), lo)
                return carry

            lax.fori_loop(0, _CHUNK // (_L * _UNROLL), body, 0)
            out_copies.append(pltpu.async_copy(
                buf.at[pl.ds(off, _CHUNK)],
                out_hbm.at[pl.ds(base + off, _CHUNK)], osem))
        for cp in out_copies:
            cp.wait()

    return sc_clamp


_sc_clamp = _build_sc_clamp()


def kernel(tensor_input, values):
    out = _sc_clamp(tensor_input.reshape(_N), values)
    return out.reshape(tensor_input.shape)
